# baseline (speedup 1.0000x reference)
import functools

import jax
import jax.numpy as jnp
from jax import lax
from jax.experimental import pallas as pl
from jax.experimental.pallas import tpu as pltpu
from jax.experimental.pallas import tpu_sc as plsc

N = 10000
E = 160000
D = 128
R = 200
NB = 10
L = 3

NC = 2
NS = 16
NW = NC * NS

NP = 10240
KN = 12288
EP = 163840
STRIPE = NP // NS

_mesh = plsc.VectorSubcoreMesh(
    core_axis_name="c", subcore_axis_name="s", num_cores=NC, num_subcores=NS)


def _wid():
    return lax.axis_index("s") * NC + lax.axis_index("c")


def _make_gather(drow, k_total, out_dtype=jnp.float32):
    kt = k_total // NW
    nq = kt // 128

    @functools.partial(
        pl.kernel,
        out_type=jax.ShapeDtypeStruct((k_total, drow), out_dtype),
        mesh=_mesh,
        scratch_types=[
            pltpu.VMEM((kt,), jnp.int32),
            pltpu.VMEM((128, drow), out_dtype),
            pltpu.VMEM((128, drow), out_dtype),
            pltpu.SemaphoreType.DMA,
            pltpu.SemaphoreType.DMA,
        ],
    )
    def gather_kernel(table_hbm, idx_hbm, out_hbm, idx_v, val0, val1, sem0, sem1):
        base_q = _wid() * nq
        pltpu.sync_copy(
            idx_hbm.at[pl.ds(pl.multiple_of(base_q * 128, 128), kt)], idx_v)
        bufs = (val0, val1)
        sems = (sem0, sem1)
        pltpu.async_copy(table_hbm.at[idx_v.at[pl.ds(0, 128)]], bufs[0], sems[0])

        def body(q, _):
            slot = lax.rem(q, 2)

            def inner(ss):
                buf, sem, nbuf, nsem = (
                    (bufs[0], sems[0], bufs[1], sems[1]) if ss == 0
                    else (bufs[1], sems[1], bufs[0], sems[0]))
                pltpu.make_async_copy(
                    table_hbm.at[idx_v.at[pl.ds(0, 128)]], buf, sem).wait()

                @pl.when(q + 1 < nq)
                def _():
                    pltpu.async_copy(
                        table_hbm.at[idx_v.at[pl.ds(
                            pl.multiple_of((q + 1) * 128, 128), 128)]],
                        nbuf, nsem)

                pltpu.sync_copy(
                    buf,
                    out_hbm.at[pl.ds(
                        pl.multiple_of((base_q + q) * 128, 128), 128)])

            @pl.when(slot == 0)
            def _():
                inner(0)

            @pl.when(slot == 1)
            def _():
                inner(1)
            return 0

        lax.fori_loop(0, nq, body, 0)

    return gather_kernel


def _make_prelude():
    kt = KN // NW
    nq = kt // 128
    et = EP // NW

    @functools.partial(
        pl.kernel,
        out_type=(
            jax.ShapeDtypeStruct((KN, D), jnp.float32),
            jax.ShapeDtypeStruct((KN, D), jnp.float32),
            jax.ShapeDtypeStruct((NW * NP,), jnp.float32),
        ),
        mesh=_mesh,
        scratch_types=[
            pltpu.VMEM((kt,), jnp.int32),
            pltpu.VMEM((kt,), jnp.int32),
            pltpu.VMEM((et,), jnp.int32),
            pltpu.VMEM((128, D), jnp.float32),
            pltpu.VMEM((128, D), jnp.float32),
            pltpu.VMEM((NP,), jnp.float32),
            pltpu.SemaphoreType.DMA,
            pltpu.SemaphoreType.DMA,
        ],
        compiler_params=pltpu.CompilerParams(needs_layout_passes=False),
    )
    def prelude_kernel(ent_hbm, ids_e_hbm, time_hbm, ids_t_hbm, dst_hbm,
                       e_out, t_out, deg_out,
                       ide_v, idt_v, dst_v, val0, val1, deg_v, sem0, sem1):
        wid = _wid()
        base = pl.multiple_of(wid * kt, 128)
        pltpu.sync_copy(ids_e_hbm.at[pl.ds(base, kt)], ide_v)
        pltpu.sync_copy(ids_t_hbm.at[pl.ds(base, kt)], idt_v)
        pltpu.sync_copy(
            dst_hbm.at[pl.ds(pl.multiple_of(wid * et, 128), et)], dst_v)

        bufs = (val0, val1)
        sems = (sem0, sem1)
        work = []
        for q in range(nq):
            work.append((ent_hbm, ide_v, e_out, q))
        for q in range(nq):
            work.append((time_hbm, idt_v, t_out, q))

        def fire(w, slot):
            tab, idx, _, q = w
            pltpu.async_copy(tab.at[idx.at[pl.ds(q * 128, 128)]],
                             bufs[slot], sems[slot])

        fire(work[0], 0)
        for i, w in enumerate(work):
            slot = i % 2
            tab, idx, out, q = w
            pltpu.make_async_copy(
                tab.at[idx.at[pl.ds(q * 128, 128)]],
                bufs[slot], sems[slot]).wait()
            if i + 1 < len(work):
                fire(work[i + 1], 1 - slot)
            pltpu.sync_copy(
                bufs[slot],
                out.at[pl.ds(pl.multiple_of(base + q * 128, 128), 128)])

        def zbody(i, _):
            deg_v[pl.ds(pl.multiple_of(i * 16, 16), 16)] = (
                jnp.zeros((16,), jnp.float32))
            return 0

        lax.fori_loop(0, NP // 16, zbody, 0)
        ones16 = jnp.ones((16,), jnp.float32)

        def dbody(i, _):
            idx16 = dst_v[pl.ds(pl.multiple_of(i * 16, 16), 16)]
            plsc.addupdate_scatter(deg_v, [idx16], ones16)
            return 0

        lax.fori_loop(0, et // 16, dbody, 0)
        pltpu.sync_copy(
            deg_v,
            deg_out.at[pl.ds(pl.multiple_of(wid * NP, 128), NP)])

    return prelude_kernel


def _make_scatter(drow):
    kt = EP // NW
    nq = kt // 128

    nb = 2
    ng = nq // nb

    @functools.partial(
        pl.kernel,
        out_type=jax.ShapeDtypeStruct((NC, NP, drow), jnp.float32),
        mesh=_mesh,
        scratch_types=[
            pltpu.VMEM_SHARED((NP, drow), jnp.float32),
            pltpu.VMEM((nq, 128), jnp.int32),
            pltpu.VMEM((128, drow), jnp.float32),
            pltpu.VMEM((128, drow), jnp.float32),
            pltpu.SemaphoreType.DMA,
            pltpu.SemaphoreType.DMA,
            pltpu.SemaphoreType.DMA,
            pltpu.SemaphoreType.DMA,
        ],
    )
    def scatter_kernel(vals_hbm, idx_hbm, zeros_hbm, out_hbm,
                       acc_sh, idx_v, b0, b1,
                       l0, l1, s0, s1):
        bufs = (b0, b1)
        lsems = (l0, l1)
        ssems = (s0, s1)
        c = lax.axis_index("c")
        s = lax.axis_index("s")
        stripe0 = pl.multiple_of(s * STRIPE, STRIPE)
        pltpu.sync_copy(zeros_hbm.at[pl.ds(stripe0, STRIPE)],
                        acc_sh.at[pl.ds(stripe0, STRIPE)])
        base_q = _wid() * nq
        pltpu.sync_copy(
            idx_hbm.at[pl.ds(pl.multiple_of(base_q, nq), nq)], idx_v)
        plsc.subcore_barrier()

        def load(q, j):
            pltpu.async_copy(
                vals_hbm.at[pl.ds(pl.multiple_of((base_q + q) * 128, 128),
                                  128)],
                bufs[j], lsems[j])

        def wait_load(j):
            pltpu.make_async_copy(
                vals_hbm.at[pl.ds(pl.multiple_of(base_q * 128, 128), 128)],
                bufs[j], lsems[j]).wait()

        def wait_scat(j):
            pltpu.make_async_copy(
                bufs[j], acc_sh.at[idx_v.at[0]], ssems[j]).wait()

        for j in range(nb):
            load(j, j)

        def body(g, _):
            for j in range(nb):
                wait_load(j)
                pltpu.async_copy(bufs[j], acc_sh.at[idx_v.at[g * nb + j]],
                                 ssems[j], add=True)
            for j in range(nb):

                @pl.when(g + 1 < ng)
                def _(j=j):
                    wait_scat(j)
                    load((g + 1) * nb + j, j)
            return 0

        lax.fori_loop(0, ng, body, 0)
        for j in range(nb):
            wait_scat(j)
        plsc.subcore_barrier()
        pltpu.sync_copy(acc_sh.at[pl.ds(stripe0, STRIPE)],
                        out_hbm.at[c, pl.ds(stripe0, STRIPE)])

    return scatter_kernel


def _fusion_body(e_ref, t_ref, w1_ref, w2_ref, b_ref, o_ref):
    o_ref[...] = jax.nn.relu(
        jnp.dot(e_ref[...], w1_ref[...], preferred_element_type=jnp.float32)
        + jnp.dot(t_ref[...], w2_ref[...], preferred_element_type=jnp.float32)
        + b_ref[...])


def _fusion(e_rows, t_rows, w1, w2, b):
    grid = NP // STRIPE
    return pl.pallas_call(
        _fusion_body,
        grid=(grid,),
        in_specs=[
            pl.BlockSpec((STRIPE, D), lambda i: (i, 0)),
            pl.BlockSpec((STRIPE, D), lambda i: (i, 0)),
            pl.BlockSpec((D, D), lambda i: (0, 0)),
            pl.BlockSpec((D, D), lambda i: (0, 0)),
            pl.BlockSpec((1, D), lambda i: (0, 0)),
        ],
        out_specs=pl.BlockSpec((STRIPE, D), lambda i: (i, 0)),
        out_shape=jax.ShapeDtypeStruct((NP, D), jnp.float32),
    )(e_rows, t_rows, w1, w2, b)


def _make_msg(l0):
    eblk = 1024
    rtab = R + 8

    def body(hs_ref, rel_ref, ctab_ref, v_ref, o_ref):
        hs = hs_ref[...]
        rel = rel_ref[0, 0, :]
        onehot = (rel[:, None]
                  == lax.broadcasted_iota(jnp.int32, (eblk, rtab), 1)
                  ).astype(jnp.float32)
        w = jnp.dot(onehot, ctab_ref[...], preferred_element_type=jnp.float32)
        acc = jnp.zeros((eblk, D), jnp.float32)
        for b in range(NB):
            p = jnp.dot(hs, v_ref[b], preferred_element_type=jnp.float32)
            acc = acc + w[:, l0 + b][:, None] * p
        o_ref[...] = acc

    def msg(hs, rel3, ctab, v_l):
        return pl.pallas_call(
            body,
            grid=(EP // eblk,),
            in_specs=[
                pl.BlockSpec((eblk, D), lambda i: (i, 0)),
                pl.BlockSpec((1, 1, eblk), lambda i: (i, 0, 0)),
                pl.BlockSpec((rtab, 32), lambda i: (0, 0)),
                pl.BlockSpec((NB, D, D), lambda i: (0, 0, 0)),
            ],
            out_specs=pl.BlockSpec((eblk, D), lambda i: (i, 0)),
            out_shape=jax.ShapeDtypeStruct((EP, D), jnp.float32),
        )(hs, rel3, ctab, v_l)

    return msg


def _upd_body(agg_ref, deg_ref, h_ref, ws_ref, b_ref, o_ref):
    deg = jnp.sum(deg_ref[...], axis=0)[:, None]
    norm = 1.0 / jnp.maximum(deg, 1.0)
    agg = agg_ref[0] + agg_ref[1]
    o_ref[...] = jax.nn.relu(
        norm * agg
        + jnp.dot(h_ref[...], ws_ref[...], preferred_element_type=jnp.float32)
        + b_ref[...])


def _update(agg2, deg2, h, w_self, b):
    grid = NP // STRIPE
    return pl.pallas_call(
        _upd_body,
        grid=(grid,),
        in_specs=[
            pl.BlockSpec((NC, STRIPE, D), lambda i: (0, i, 0)),
            pl.BlockSpec((NW, STRIPE), lambda i: (0, i)),
            pl.BlockSpec((STRIPE, D), lambda i: (i, 0)),
            pl.BlockSpec((D, D), lambda i: (0, 0)),
            pl.BlockSpec((1, D), lambda i: (0, 0)),
        ],
        out_specs=pl.BlockSpec((STRIPE, D), lambda i: (i, 0)),
        out_shape=jax.ShapeDtypeStruct((NP, D), jnp.float32),
    )(agg2, deg2, h, w_self, b)


def kernel(entity_ids, rel_ids, time_ids, edge_index, entity_table, time_table,
           fusion_W, fusion_b, V, comp, W_self, layer_bias):
    src = edge_index[0]
    dst = edge_index[1]

    ids_e = jnp.concatenate([entity_ids, jnp.zeros((KN - N,), jnp.int32)])
    ids_t = jnp.concatenate([time_ids, jnp.zeros((KN - N,), jnp.int32)])
    srcp = jnp.concatenate([src, jnp.zeros((EP - E,), jnp.int32)])
    dstp = jnp.concatenate(
        [dst, jnp.full((EP - E,), N, jnp.int32)]).reshape(EP // 128, 128)
    relp = jnp.concatenate([rel_ids, jnp.full((EP - E,), R, jnp.int32)])

    ctab = comp.transpose(1, 0, 2).reshape(R, L * NB)
    ctab = jnp.pad(ctab, ((0, 8), (0, 32 - L * NB)))

    z128 = jnp.zeros((NP, D), jnp.float32)
    rel3 = relp.reshape(EP // 1024, 1, 1024)
    dstf = jnp.concatenate([dst, jnp.full((EP - E,), N, jnp.int32)])
    w1 = fusion_W[:D]
    w2 = fusion_W[D:]
    bias0 = fusion_b.reshape(1, D)

    e_rows, t_rows, degf = _make_prelude()(
        entity_table, ids_e, time_table, ids_t, dstf)
    degp = degf.reshape(NW, NP)
    scatter_m = _make_scatter(D)

    h = _fusion(e_rows[:NP], t_rows[:NP], w1, w2, bias0)

    gather_h = _make_gather(D, EP)
    for l in range(L):
        hs = gather_h(h, srcp)
        m = _make_msg(l * NB)(hs, rel3, ctab, V[l])
        agg2 = scatter_m(m, dstp, z128)
        h = _update(agg2, degp, h, W_self[l], layer_bias[l].reshape(1, D))

    return h[:N]


# speedup vs baseline: 1.4856x; 1.4856x over previous
import functools

import jax
import jax.numpy as jnp
from jax import lax
from jax.experimental import pallas as pl
from jax.experimental.pallas import tpu as pltpu
from jax.experimental.pallas import tpu_sc as plsc

N = 10000
E = 160000
D = 128
R = 200
NB = 10
L = 3

NC = 2
NS = 16
NW = NC * NS

NP = 10240
KN = 12288
EP = 163840
STRIPE = NP // NS

_mesh = plsc.VectorSubcoreMesh(
    core_axis_name="c", subcore_axis_name="s", num_cores=NC, num_subcores=NS)


def _wid():
    return lax.axis_index("s") * NC + lax.axis_index("c")


def _make_gather(drow, k_total, out_dtype=jnp.float32):
    kt = k_total // NW
    nq = kt // 128

    @functools.partial(
        pl.kernel,
        out_type=jax.ShapeDtypeStruct((k_total, drow), out_dtype),
        mesh=_mesh,
        scratch_types=[
            pltpu.VMEM((kt,), jnp.int32),
            pltpu.VMEM((128, drow), out_dtype),
            pltpu.VMEM((128, drow), out_dtype),
            pltpu.SemaphoreType.DMA,
            pltpu.SemaphoreType.DMA,
        ],
    )
    def gather_kernel(table_hbm, idx_hbm, out_hbm, idx_v, val0, val1, sem0, sem1):
        base_q = _wid() * nq
        pltpu.sync_copy(
            idx_hbm.at[pl.ds(pl.multiple_of(base_q * 128, 128), kt)], idx_v)
        bufs = (val0, val1)
        sems = (sem0, sem1)
        pltpu.async_copy(table_hbm.at[idx_v.at[pl.ds(0, 128)]], bufs[0], sems[0])

        def body(q, _):
            slot = lax.rem(q, 2)

            def inner(ss):
                buf, sem, nbuf, nsem = (
                    (bufs[0], sems[0], bufs[1], sems[1]) if ss == 0
                    else (bufs[1], sems[1], bufs[0], sems[0]))
                pltpu.make_async_copy(
                    table_hbm.at[idx_v.at[pl.ds(0, 128)]], buf, sem).wait()

                @pl.when(q + 1 < nq)
                def _():
                    pltpu.async_copy(
                        table_hbm.at[idx_v.at[pl.ds(
                            pl.multiple_of((q + 1) * 128, 128), 128)]],
                        nbuf, nsem)

                pltpu.sync_copy(
                    buf,
                    out_hbm.at[pl.ds(
                        pl.multiple_of((base_q + q) * 128, 128), 128)])

            @pl.when(slot == 0)
            def _():
                inner(0)

            @pl.when(slot == 1)
            def _():
                inner(1)
            return 0

        lax.fori_loop(0, nq, body, 0)

    return gather_kernel


def _make_gather_sp(k_total):
    kt = k_total // NW
    nq = kt // 128

    @functools.partial(
        pl.kernel,
        out_type=jax.ShapeDtypeStruct((k_total, D), jnp.float32),
        mesh=_mesh,
        scratch_types=[
            pltpu.VMEM_SHARED((NP, D), jnp.float32),
            pltpu.VMEM((kt,), jnp.int32),
            pltpu.VMEM((128, D), jnp.float32),
            pltpu.VMEM((128, D), jnp.float32),
            pltpu.SemaphoreType.DMA,
            pltpu.SemaphoreType.DMA,
        ],
    )
    def gather_kernel(table_hbm, idx_hbm, out_hbm,
                      tab_sh, idx_v, val0, val1, sem0, sem1):
        s = lax.axis_index("s")
        stripe0 = pl.multiple_of(s * STRIPE, STRIPE)
        pltpu.sync_copy(table_hbm.at[pl.ds(stripe0, STRIPE)],
                        tab_sh.at[pl.ds(stripe0, STRIPE)])
        base_q = _wid() * nq
        pltpu.sync_copy(
            idx_hbm.at[pl.ds(pl.multiple_of(base_q * 128, 128), kt)], idx_v)
        plsc.subcore_barrier()

        bufs = (val0, val1)
        sems = (sem0, sem1)
        pltpu.async_copy(tab_sh.at[idx_v.at[pl.ds(0, 128)]], bufs[0], sems[0])

        def body(q, _):
            slot = lax.rem(q, 2)

            def inner(ss):
                buf, sem, nbuf, nsem = (
                    (bufs[0], sems[0], bufs[1], sems[1]) if ss == 0
                    else (bufs[1], sems[1], bufs[0], sems[0]))
                pltpu.make_async_copy(
                    tab_sh.at[idx_v.at[pl.ds(0, 128)]], buf, sem).wait()

                @pl.when(q + 1 < nq)
                def _():
                    pltpu.async_copy(
                        tab_sh.at[idx_v.at[pl.ds(
                            pl.multiple_of((q + 1) * 128, 128), 128)]],
                        nbuf, nsem)

                pltpu.sync_copy(
                    buf,
                    out_hbm.at[pl.ds(
                        pl.multiple_of((base_q + q) * 128, 128), 128)])

            @pl.when(slot == 0)
            def _():
                inner(0)

            @pl.when(slot == 1)
            def _():
                inner(1)
            return 0

        lax.fori_loop(0, nq, body, 0)

    return gather_kernel


def _make_prelude():
    kt = KN // NW
    nq = kt // 128
    et = EP // NW

    @functools.partial(
        pl.kernel,
        out_type=(
            jax.ShapeDtypeStruct((KN, D), jnp.float32),
            jax.ShapeDtypeStruct((KN, D), jnp.float32),
            jax.ShapeDtypeStruct((NW * NP,), jnp.float32),
        ),
        mesh=_mesh,
        scratch_types=[
            pltpu.VMEM((kt,), jnp.int32),
            pltpu.VMEM((kt,), jnp.int32),
            pltpu.VMEM((et,), jnp.int32),
            pltpu.VMEM((128, D), jnp.float32),
            pltpu.VMEM((128, D), jnp.float32),
            pltpu.VMEM((NP,), jnp.float32),
            pltpu.SemaphoreType.DMA,
            pltpu.SemaphoreType.DMA,
        ],
        compiler_params=pltpu.CompilerParams(needs_layout_passes=False),
    )
    def prelude_kernel(ent_hbm, ids_e_hbm, time_hbm, ids_t_hbm, dst_hbm,
                       e_out, t_out, deg_out,
                       ide_v, idt_v, dst_v, val0, val1, deg_v, sem0, sem1):
        wid = _wid()
        base = pl.multiple_of(wid * kt, 128)
        pltpu.sync_copy(ids_e_hbm.at[pl.ds(base, kt)], ide_v)
        pltpu.sync_copy(ids_t_hbm.at[pl.ds(base, kt)], idt_v)
        pltpu.sync_copy(
            dst_hbm.at[pl.ds(pl.multiple_of(wid * et, 128), et)], dst_v)

        bufs = (val0, val1)
        sems = (sem0, sem1)
        work = []
        for q in range(nq):
            work.append((ent_hbm, ide_v, e_out, q))
        for q in range(nq):
            work.append((time_hbm, idt_v, t_out, q))

        def fire(w, slot):
            tab, idx, _, q = w
            pltpu.async_copy(tab.at[idx.at[pl.ds(q * 128, 128)]],
                             bufs[slot], sems[slot])

        fire(work[0], 0)
        for i, w in enumerate(work):
            slot = i % 2
            tab, idx, out, q = w
            pltpu.make_async_copy(
                tab.at[idx.at[pl.ds(q * 128, 128)]],
                bufs[slot], sems[slot]).wait()
            if i + 1 < len(work):
                fire(work[i + 1], 1 - slot)
            pltpu.sync_copy(
                bufs[slot],
                out.at[pl.ds(pl.multiple_of(base + q * 128, 128), 128)])

        def zbody(i, _):
            deg_v[pl.ds(pl.multiple_of(i * 16, 16), 16)] = (
                jnp.zeros((16,), jnp.float32))
            return 0

        lax.fori_loop(0, NP // 16, zbody, 0)
        ones16 = jnp.ones((16,), jnp.float32)

        def dbody(i, _):
            idx16 = dst_v[pl.ds(pl.multiple_of(i * 16, 16), 16)]
            plsc.addupdate_scatter(deg_v, [idx16], ones16)
            return 0

        lax.fori_loop(0, et // 16, dbody, 0)
        pltpu.sync_copy(
            deg_v,
            deg_out.at[pl.ds(pl.multiple_of(wid * NP, 128), NP)])

    return prelude_kernel


def _make_scatter(drow):
    kt = EP // NW
    nq = kt // 128

    nb = 2
    ng = nq // nb

    @functools.partial(
        pl.kernel,
        out_type=jax.ShapeDtypeStruct((NC, NP, drow), jnp.float32),
        mesh=_mesh,
        scratch_types=[
            pltpu.VMEM_SHARED((NP, drow), jnp.float32),
            pltpu.VMEM((nq, 128), jnp.int32),
            pltpu.VMEM((128, drow), jnp.float32),
            pltpu.VMEM((128, drow), jnp.float32),
            pltpu.SemaphoreType.DMA,
            pltpu.SemaphoreType.DMA,
            pltpu.SemaphoreType.DMA,
            pltpu.SemaphoreType.DMA,
        ],
    )
    def scatter_kernel(vals_hbm, idx_hbm, zeros_hbm, out_hbm,
                       acc_sh, idx_v, b0, b1,
                       l0, l1, s0, s1):
        bufs = (b0, b1)
        lsems = (l0, l1)
        ssems = (s0, s1)
        c = lax.axis_index("c")
        s = lax.axis_index("s")
        stripe0 = pl.multiple_of(s * STRIPE, STRIPE)
        pltpu.sync_copy(zeros_hbm.at[pl.ds(stripe0, STRIPE)],
                        acc_sh.at[pl.ds(stripe0, STRIPE)])
        base_q = _wid() * nq
        pltpu.sync_copy(
            idx_hbm.at[pl.ds(pl.multiple_of(base_q, nq), nq)], idx_v)
        plsc.subcore_barrier()

        def load(q, j):
            pltpu.async_copy(
                vals_hbm.at[pl.ds(pl.multiple_of((base_q + q) * 128, 128),
                                  128)],
                bufs[j], lsems[j])

        def wait_load(j):
            pltpu.make_async_copy(
                vals_hbm.at[pl.ds(pl.multiple_of(base_q * 128, 128), 128)],
                bufs[j], lsems[j]).wait()

        def wait_scat(j):
            pltpu.make_async_copy(
                bufs[j], acc_sh.at[idx_v.at[0]], ssems[j]).wait()

        for j in range(nb):
            load(j, j)

        def body(g, _):
            for j in range(nb):
                wait_load(j)
                pltpu.async_copy(bufs[j], acc_sh.at[idx_v.at[g * nb + j]],
                                 ssems[j], add=True)
            for j in range(nb):

                @pl.when(g + 1 < ng)
                def _(j=j):
                    wait_scat(j)
                    load((g + 1) * nb + j, j)
            return 0

        lax.fori_loop(0, ng, body, 0)
        for j in range(nb):
            wait_scat(j)
        plsc.subcore_barrier()
        pltpu.sync_copy(acc_sh.at[pl.ds(stripe0, STRIPE)],
                        out_hbm.at[c, pl.ds(stripe0, STRIPE)])

    return scatter_kernel


def _fusion_body(e_ref, t_ref, w1_ref, w2_ref, b_ref, o_ref):
    o_ref[...] = jax.nn.relu(
        jnp.dot(e_ref[...], w1_ref[...], preferred_element_type=jnp.float32)
        + jnp.dot(t_ref[...], w2_ref[...], preferred_element_type=jnp.float32)
        + b_ref[...])


def _fusion(e_rows, t_rows, w1, w2, b):
    grid = NP // STRIPE
    return pl.pallas_call(
        _fusion_body,
        grid=(grid,),
        in_specs=[
            pl.BlockSpec((STRIPE, D), lambda i: (i, 0)),
            pl.BlockSpec((STRIPE, D), lambda i: (i, 0)),
            pl.BlockSpec((D, D), lambda i: (0, 0)),
            pl.BlockSpec((D, D), lambda i: (0, 0)),
            pl.BlockSpec((1, D), lambda i: (0, 0)),
        ],
        out_specs=pl.BlockSpec((STRIPE, D), lambda i: (i, 0)),
        out_shape=jax.ShapeDtypeStruct((NP, D), jnp.float32),
    )(e_rows, t_rows, w1, w2, b)


def _make_msg(l0):
    eblk = 1024
    rtab = R + 8

    def body(hs_ref, rel_ref, ctab_ref, v_ref, o_ref):
        hs = hs_ref[...]
        rel = rel_ref[0, 0, :]
        onehot = (rel[:, None]
                  == lax.broadcasted_iota(jnp.int32, (eblk, rtab), 1)
                  ).astype(jnp.float32)
        w = jnp.dot(onehot, ctab_ref[...], preferred_element_type=jnp.float32)
        acc = jnp.zeros((eblk, D), jnp.float32)
        for b in range(NB):
            p = jnp.dot(hs, v_ref[b], preferred_element_type=jnp.float32)
            acc = acc + w[:, l0 + b][:, None] * p
        o_ref[...] = acc

    def msg(hs, rel3, ctab, v_l):
        return pl.pallas_call(
            body,
            grid=(EP // eblk,),
            in_specs=[
                pl.BlockSpec((eblk, D), lambda i: (i, 0)),
                pl.BlockSpec((1, 1, eblk), lambda i: (i, 0, 0)),
                pl.BlockSpec((rtab, 32), lambda i: (0, 0)),
                pl.BlockSpec((NB, D, D), lambda i: (0, 0, 0)),
            ],
            out_specs=pl.BlockSpec((eblk, D), lambda i: (i, 0)),
            out_shape=jax.ShapeDtypeStruct((EP, D), jnp.float32),
        )(hs, rel3, ctab, v_l)

    return msg


def _upd_body(agg_ref, deg_ref, h_ref, ws_ref, b_ref, o_ref):
    deg = jnp.sum(deg_ref[...], axis=0)[:, None]
    norm = 1.0 / jnp.maximum(deg, 1.0)
    agg = agg_ref[0] + agg_ref[1]
    o_ref[...] = jax.nn.relu(
        norm * agg
        + jnp.dot(h_ref[...], ws_ref[...], preferred_element_type=jnp.float32)
        + b_ref[...])


def _update(agg2, deg2, h, w_self, b):
    grid = NP // STRIPE
    return pl.pallas_call(
        _upd_body,
        grid=(grid,),
        in_specs=[
            pl.BlockSpec((NC, STRIPE, D), lambda i: (0, i, 0)),
            pl.BlockSpec((NW, STRIPE), lambda i: (0, i)),
            pl.BlockSpec((STRIPE, D), lambda i: (i, 0)),
            pl.BlockSpec((D, D), lambda i: (0, 0)),
            pl.BlockSpec((1, D), lambda i: (0, 0)),
        ],
        out_specs=pl.BlockSpec((STRIPE, D), lambda i: (i, 0)),
        out_shape=jax.ShapeDtypeStruct((NP, D), jnp.float32),
    )(agg2, deg2, h, w_self, b)


def kernel(entity_ids, rel_ids, time_ids, edge_index, entity_table, time_table,
           fusion_W, fusion_b, V, comp, W_self, layer_bias):
    src = edge_index[0]
    dst = edge_index[1]

    ids_e = jnp.concatenate([entity_ids, jnp.zeros((KN - N,), jnp.int32)])
    ids_t = jnp.concatenate([time_ids, jnp.zeros((KN - N,), jnp.int32)])
    srcp = jnp.concatenate([src, jnp.zeros((EP - E,), jnp.int32)])
    dstp = jnp.concatenate(
        [dst, jnp.full((EP - E,), N, jnp.int32)]).reshape(EP // 128, 128)
    relp = jnp.concatenate([rel_ids, jnp.full((EP - E,), R, jnp.int32)])

    ctab = comp.transpose(1, 0, 2).reshape(R, L * NB)
    ctab = jnp.pad(ctab, ((0, 8), (0, 32 - L * NB)))

    z128 = jnp.zeros((NP, D), jnp.float32)
    rel3 = relp.reshape(EP // 1024, 1, 1024)
    dstf = jnp.concatenate([dst, jnp.full((EP - E,), N, jnp.int32)])
    w1 = fusion_W[:D]
    w2 = fusion_W[D:]
    bias0 = fusion_b.reshape(1, D)

    e_rows, t_rows, degf = _make_prelude()(
        entity_table, ids_e, time_table, ids_t, dstf)
    degp = degf.reshape(NW, NP)
    scatter_m = _make_scatter(D)

    h = _fusion(e_rows[:NP], t_rows[:NP], w1, w2, bias0)

    gather_h = _make_gather_sp(EP)
    for l in range(L):
        hs = gather_h(h, srcp)
        m = _make_msg(l * NB)(hs, rel3, ctab, V[l])
        agg2 = scatter_m(m, dstp, z128)
        h = _update(agg2, degp, h, W_self[l], layer_bias[l].reshape(1, D))

    return h[:N]


# speedup vs baseline: 1.4967x; 1.0074x over previous
import functools

import jax
import jax.numpy as jnp
from jax import lax
from jax.experimental import pallas as pl
from jax.experimental.pallas import tpu as pltpu
from jax.experimental.pallas import tpu_sc as plsc

N = 10000
E = 160000
D = 128
R = 200
NB = 10
L = 3

NC = 2
NS = 16
NW = NC * NS

NP = 10240
KN = 12288
EP = 163840
STRIPE = NP // NS

_mesh = plsc.VectorSubcoreMesh(
    core_axis_name="c", subcore_axis_name="s", num_cores=NC, num_subcores=NS)


def _wid():
    return lax.axis_index("s") * NC + lax.axis_index("c")


def _make_gather(drow, k_total, out_dtype=jnp.float32):
    kt = k_total // NW
    nq = kt // 128

    @functools.partial(
        pl.kernel,
        out_type=jax.ShapeDtypeStruct((k_total, drow), out_dtype),
        mesh=_mesh,
        scratch_types=[
            pltpu.VMEM((kt,), jnp.int32),
            pltpu.VMEM((128, drow), out_dtype),
            pltpu.VMEM((128, drow), out_dtype),
            pltpu.SemaphoreType.DMA,
            pltpu.SemaphoreType.DMA,
        ],
    )
    def gather_kernel(table_hbm, idx_hbm, out_hbm, idx_v, val0, val1, sem0, sem1):
        base_q = _wid() * nq
        pltpu.sync_copy(
            idx_hbm.at[pl.ds(pl.multiple_of(base_q * 128, 128), kt)], idx_v)
        bufs = (val0, val1)
        sems = (sem0, sem1)
        pltpu.async_copy(table_hbm.at[idx_v.at[pl.ds(0, 128)]], bufs[0], sems[0])

        def body(q, _):
            slot = lax.rem(q, 2)

            def inner(ss):
                buf, sem, nbuf, nsem = (
                    (bufs[0], sems[0], bufs[1], sems[1]) if ss == 0
                    else (bufs[1], sems[1], bufs[0], sems[0]))
                pltpu.make_async_copy(
                    table_hbm.at[idx_v.at[pl.ds(0, 128)]], buf, sem).wait()

                @pl.when(q + 1 < nq)
                def _():
                    pltpu.async_copy(
                        table_hbm.at[idx_v.at[pl.ds(
                            pl.multiple_of((q + 1) * 128, 128), 128)]],
                        nbuf, nsem)

                pltpu.sync_copy(
                    buf,
                    out_hbm.at[pl.ds(
                        pl.multiple_of((base_q + q) * 128, 128), 128)])

            @pl.when(slot == 0)
            def _():
                inner(0)

            @pl.when(slot == 1)
            def _():
                inner(1)
            return 0

        lax.fori_loop(0, nq, body, 0)

    return gather_kernel


def _make_gather_sp(k_total):
    kt = k_total // NW
    nq = kt // 128

    @functools.partial(
        pl.kernel,
        out_type=jax.ShapeDtypeStruct((k_total, D), jnp.float32),
        mesh=_mesh,
        scratch_types=[
            pltpu.VMEM_SHARED((NP, D), jnp.float32),
            pltpu.VMEM((kt,), jnp.int32),
            pltpu.VMEM((128, D), jnp.float32),
            pltpu.VMEM((128, D), jnp.float32),
            pltpu.SemaphoreType.DMA,
            pltpu.SemaphoreType.DMA,
        ],
    )
    def gather_kernel(table_hbm, idx_hbm, out_hbm,
                      tab_sh, idx_v, val0, val1, sem0, sem1):
        s = lax.axis_index("s")
        stripe0 = pl.multiple_of(s * STRIPE, STRIPE)
        pltpu.sync_copy(table_hbm.at[pl.ds(stripe0, STRIPE)],
                        tab_sh.at[pl.ds(stripe0, STRIPE)])
        base_q = _wid() * nq
        pltpu.sync_copy(
            idx_hbm.at[pl.ds(pl.multiple_of(base_q * 128, 128), kt)], idx_v)
        plsc.subcore_barrier()

        bufs = (val0, val1)
        sems = (sem0, sem1)
        pltpu.async_copy(tab_sh.at[idx_v.at[pl.ds(0, 128)]], bufs[0], sems[0])

        def body(q, _):
            slot = lax.rem(q, 2)

            def inner(ss):
                buf, sem, nbuf, nsem = (
                    (bufs[0], sems[0], bufs[1], sems[1]) if ss == 0
                    else (bufs[1], sems[1], bufs[0], sems[0]))
                pltpu.make_async_copy(
                    tab_sh.at[idx_v.at[pl.ds(0, 128)]], buf, sem).wait()

                @pl.when(q + 1 < nq)
                def _():
                    pltpu.async_copy(
                        tab_sh.at[idx_v.at[pl.ds(
                            pl.multiple_of((q + 1) * 128, 128), 128)]],
                        nbuf, nsem)

                pltpu.sync_copy(
                    buf,
                    out_hbm.at[pl.ds(
                        pl.multiple_of((base_q + q) * 128, 128), 128)])

            @pl.when(slot == 0)
            def _():
                inner(0)

            @pl.when(slot == 1)
            def _():
                inner(1)
            return 0

        lax.fori_loop(0, nq, body, 0)

    return gather_kernel


def _make_prelude():
    kt = KN // NW
    nq = kt // 128
    et = EP // NW

    @functools.partial(
        pl.kernel,
        out_type=(
            jax.ShapeDtypeStruct((KN, D), jnp.float32),
            jax.ShapeDtypeStruct((KN, D), jnp.float32),
            jax.ShapeDtypeStruct((NW * NP,), jnp.float32),
        ),
        mesh=_mesh,
        scratch_types=[
            pltpu.VMEM((kt,), jnp.int32),
            pltpu.VMEM((kt,), jnp.int32),
            pltpu.VMEM((et,), jnp.int32),
            pltpu.VMEM((128, D), jnp.float32),
            pltpu.VMEM((128, D), jnp.float32),
            pltpu.VMEM((NP,), jnp.float32),
            pltpu.SemaphoreType.DMA,
            pltpu.SemaphoreType.DMA,
        ],
        compiler_params=pltpu.CompilerParams(needs_layout_passes=False),
    )
    def prelude_kernel(ent_hbm, ids_e_hbm, time_hbm, ids_t_hbm, dst_hbm,
                       e_out, t_out, deg_out,
                       ide_v, idt_v, dst_v, val0, val1, deg_v, sem0, sem1):
        wid = _wid()
        base = pl.multiple_of(wid * kt, 128)
        pltpu.sync_copy(ids_e_hbm.at[pl.ds(base, kt)], ide_v)
        pltpu.sync_copy(ids_t_hbm.at[pl.ds(base, kt)], idt_v)
        pltpu.sync_copy(
            dst_hbm.at[pl.ds(pl.multiple_of(wid * et, 128), et)], dst_v)

        bufs = (val0, val1)
        sems = (sem0, sem1)
        work = []
        for q in range(nq):
            work.append((ent_hbm, ide_v, e_out, q))
        for q in range(nq):
            work.append((time_hbm, idt_v, t_out, q))

        def fire(w, slot):
            tab, idx, _, q = w
            pltpu.async_copy(tab.at[idx.at[pl.ds(q * 128, 128)]],
                             bufs[slot], sems[slot])

        fire(work[0], 0)
        for i, w in enumerate(work):
            slot = i % 2
            tab, idx, out, q = w
            pltpu.make_async_copy(
                tab.at[idx.at[pl.ds(q * 128, 128)]],
                bufs[slot], sems[slot]).wait()
            if i + 1 < len(work):
                fire(work[i + 1], 1 - slot)
            pltpu.sync_copy(
                bufs[slot],
                out.at[pl.ds(pl.multiple_of(base + q * 128, 128), 128)])

        def zbody(i, _):
            deg_v[pl.ds(pl.multiple_of(i * 16, 16), 16)] = (
                jnp.zeros((16,), jnp.float32))
            return 0

        lax.fori_loop(0, NP // 16, zbody, 0)
        ones16 = jnp.ones((16,), jnp.float32)

        def dbody(i, _):
            idx16 = dst_v[pl.ds(pl.multiple_of(i * 16, 16), 16)]
            plsc.addupdate_scatter(deg_v, [idx16], ones16)
            return 0

        lax.fori_loop(0, et // 16, dbody, 0)
        pltpu.sync_copy(
            deg_v,
            deg_out.at[pl.ds(pl.multiple_of(wid * NP, 128), NP)])

    return prelude_kernel


def _make_scatter(drow):
    kt = EP // NW
    nq = kt // 128

    nb = 2
    ng = nq // nb

    @functools.partial(
        pl.kernel,
        out_type=jax.ShapeDtypeStruct((NC, NP, drow), jnp.float32),
        mesh=_mesh,
        scratch_types=[
            pltpu.VMEM_SHARED((NP, drow), jnp.float32),
            pltpu.VMEM((nq, 128), jnp.int32),
            pltpu.VMEM((128, drow), jnp.float32),
            pltpu.VMEM((128, drow), jnp.float32),
            pltpu.SemaphoreType.DMA,
            pltpu.SemaphoreType.DMA,
            pltpu.SemaphoreType.DMA,
            pltpu.SemaphoreType.DMA,
        ],
    )
    def scatter_kernel(vals_hbm, idx_hbm, zeros_hbm, out_hbm,
                       acc_sh, idx_v, b0, b1,
                       l0, l1, s0, s1):
        bufs = (b0, b1)
        lsems = (l0, l1)
        ssems = (s0, s1)
        c = lax.axis_index("c")
        s = lax.axis_index("s")
        stripe0 = pl.multiple_of(s * STRIPE, STRIPE)
        pltpu.sync_copy(zeros_hbm.at[pl.ds(stripe0, STRIPE)],
                        acc_sh.at[pl.ds(stripe0, STRIPE)])
        base_q = _wid() * nq
        pltpu.sync_copy(
            idx_hbm.at[pl.ds(pl.multiple_of(base_q, nq), nq)], idx_v)
        plsc.subcore_barrier()

        def load(q, j):
            pltpu.async_copy(
                vals_hbm.at[pl.ds(pl.multiple_of((base_q + q) * 128, 128),
                                  128)],
                bufs[j], lsems[j])

        def wait_load(j):
            pltpu.make_async_copy(
                vals_hbm.at[pl.ds(pl.multiple_of(base_q * 128, 128), 128)],
                bufs[j], lsems[j]).wait()

        def wait_scat(j):
            pltpu.make_async_copy(
                bufs[j], acc_sh.at[idx_v.at[0]], ssems[j]).wait()

        for j in range(nb):
            load(j, j)

        def body(g, _):
            for j in range(nb):
                wait_load(j)
                pltpu.async_copy(bufs[j], acc_sh.at[idx_v.at[g * nb + j]],
                                 ssems[j], add=True)
            for j in range(nb):

                @pl.when(g + 1 < ng)
                def _(j=j):
                    wait_scat(j)
                    load((g + 1) * nb + j, j)
            return 0

        lax.fori_loop(0, ng, body, 0)
        for j in range(nb):
            wait_scat(j)
        plsc.subcore_barrier()
        pltpu.sync_copy(acc_sh.at[pl.ds(stripe0, STRIPE)],
                        out_hbm.at[c, pl.ds(stripe0, STRIPE)])

    return scatter_kernel


def _fusion_body(e_ref, t_ref, w1_ref, w2_ref, b_ref, o_ref):
    o_ref[...] = jax.nn.relu(
        jnp.dot(e_ref[...], w1_ref[...], preferred_element_type=jnp.float32)
        + jnp.dot(t_ref[...], w2_ref[...], preferred_element_type=jnp.float32)
        + b_ref[...])


def _fusion(e_rows, t_rows, w1, w2, b):
    grid = NP // STRIPE
    return pl.pallas_call(
        _fusion_body,
        grid=(grid,),
        in_specs=[
            pl.BlockSpec((STRIPE, D), lambda i: (i, 0)),
            pl.BlockSpec((STRIPE, D), lambda i: (i, 0)),
            pl.BlockSpec((D, D), lambda i: (0, 0)),
            pl.BlockSpec((D, D), lambda i: (0, 0)),
            pl.BlockSpec((1, D), lambda i: (0, 0)),
        ],
        out_specs=pl.BlockSpec((STRIPE, D), lambda i: (i, 0)),
        out_shape=jax.ShapeDtypeStruct((NP, D), jnp.float32),
    )(e_rows, t_rows, w1, w2, b)


def _make_msg(l0):
    eblk = 1024
    rtab = R + 8

    def body(hs_ref, rel_ref, ctab_ref, v_ref, o_ref):
        hs = hs_ref[...]
        rel = rel_ref[0, 0, :]
        onehot = (rel[:, None]
                  == lax.broadcasted_iota(jnp.int32, (eblk, rtab), 1)
                  ).astype(jnp.float32)
        w = jnp.dot(onehot, ctab_ref[...], preferred_element_type=jnp.float32)
        hs_bf = hs.astype(jnp.bfloat16)
        acc = jnp.zeros((eblk, D), jnp.float32)
        for b in range(NB):
            p = jnp.dot(hs_bf, v_ref[b], preferred_element_type=jnp.float32)
            acc = acc + w[:, l0 + b][:, None] * p
        o_ref[...] = acc

    def msg(hs, rel3, ctab, v_l):
        return pl.pallas_call(
            body,
            grid=(EP // eblk,),
            in_specs=[
                pl.BlockSpec((eblk, D), lambda i: (i, 0)),
                pl.BlockSpec((1, 1, eblk), lambda i: (i, 0, 0)),
                pl.BlockSpec((rtab, 32), lambda i: (0, 0)),
                pl.BlockSpec((NB, D, D), lambda i: (0, 0, 0)),
            ],
            out_specs=pl.BlockSpec((eblk, D), lambda i: (i, 0)),
            out_shape=jax.ShapeDtypeStruct((EP, D), jnp.float32),
        )(hs, rel3, ctab, v_l.astype(jnp.bfloat16))

    return msg


def _upd_body(agg_ref, deg_ref, h_ref, ws_ref, b_ref, o_ref):
    deg = jnp.sum(deg_ref[...], axis=0)[:, None]
    norm = 1.0 / jnp.maximum(deg, 1.0)
    agg = agg_ref[0] + agg_ref[1]
    o_ref[...] = jax.nn.relu(
        norm * agg
        + jnp.dot(h_ref[...], ws_ref[...], preferred_element_type=jnp.float32)
        + b_ref[...])


def _update(agg2, deg2, h, w_self, b):
    grid = NP // STRIPE
    return pl.pallas_call(
        _upd_body,
        grid=(grid,),
        in_specs=[
            pl.BlockSpec((NC, STRIPE, D), lambda i: (0, i, 0)),
            pl.BlockSpec((NW, STRIPE), lambda i: (0, i)),
            pl.BlockSpec((STRIPE, D), lambda i: (i, 0)),
            pl.BlockSpec((D, D), lambda i: (0, 0)),
            pl.BlockSpec((1, D), lambda i: (0, 0)),
        ],
        out_specs=pl.BlockSpec((STRIPE, D), lambda i: (i, 0)),
        out_shape=jax.ShapeDtypeStruct((NP, D), jnp.float32),
    )(agg2, deg2, h, w_self, b)


def kernel(entity_ids, rel_ids, time_ids, edge_index, entity_table, time_table,
           fusion_W, fusion_b, V, comp, W_self, layer_bias):
    src = edge_index[0]
    dst = edge_index[1]

    ids_e = jnp.concatenate([entity_ids, jnp.zeros((KN - N,), jnp.int32)])
    ids_t = jnp.concatenate([time_ids, jnp.zeros((KN - N,), jnp.int32)])
    srcp = jnp.concatenate([src, jnp.zeros((EP - E,), jnp.int32)])
    dstp = jnp.concatenate(
        [dst, jnp.full((EP - E,), N, jnp.int32)]).reshape(EP // 128, 128)
    relp = jnp.concatenate([rel_ids, jnp.full((EP - E,), R, jnp.int32)])

    ctab = comp.transpose(1, 0, 2).reshape(R, L * NB)
    ctab = jnp.pad(ctab, ((0, 8), (0, 32 - L * NB)))

    z128 = jnp.zeros((NP, D), jnp.float32)
    rel3 = relp.reshape(EP // 1024, 1, 1024)
    dstf = jnp.concatenate([dst, jnp.full((EP - E,), N, jnp.int32)])
    w1 = fusion_W[:D]
    w2 = fusion_W[D:]
    bias0 = fusion_b.reshape(1, D)

    e_rows, t_rows, degf = _make_prelude()(
        entity_table, ids_e, time_table, ids_t, dstf)
    degp = degf.reshape(NW, NP)
    scatter_m = _make_scatter(D)

    h = _fusion(e_rows[:NP], t_rows[:NP], w1, w2, bias0)

    gather_h = _make_gather_sp(EP)
    for l in range(L):
        hs = gather_h(h, srcp)
        m = _make_msg(l * NB)(hs, rel3, ctab, V[l])
        agg2 = scatter_m(m, dstp, z128)
        h = _update(agg2, degp, h, W_self[l], layer_bias[l].reshape(1, D))

    return h[:N]
